# baseline (speedup 1.0000x reference)
import functools

import jax
import jax.numpy as jnp
from jax.experimental import pallas as pl


def _moe_body(n_f, h_ref, gw_ref, gb_ref, w1_ref, b1_ref, w3_ref, b3_ref,
              w2_ref, b2_ref, out_ref, logits_ref):
    f = pl.program_id(1)
    h = h_ref[...]
    dn = (((1,), (1,)), ((), ()))
    a1 = jax.lax.dot_general(h, w1_ref[...], dn,
                             preferred_element_type=jnp.float32) + b1_ref[...]
    a3 = jax.lax.dot_general(h, w3_ref[...], dn,
                             preferred_element_type=jnp.float32) + b3_ref[...]
    act = (a1 * jax.nn.sigmoid(a1)) * a3
    partial = jax.lax.dot_general(act.astype(h.dtype), w2_ref[...], dn,
                                  preferred_element_type=jnp.float32)

    @pl.when(f == 0)
    def _init():
        logits_ref[...] = jax.lax.dot_general(
            h, gw_ref[...], dn, preferred_element_type=jnp.float32
        ) + gb_ref[...]
        out_ref[...] = partial + b2_ref[...]

    @pl.when(f != 0)
    def _accum():
        out_ref[...] += partial


def _fused_moe(h, gate_w, gate_b, w1_w, w1_b, w3_w, w3_b, w2_w, w2_b,
               blk_t, blk_f):
    n_tokens, hidden = h.shape
    ffn = w1_w.shape[0]
    n_experts = gate_w.shape[0]
    n_t = n_tokens // blk_t
    n_f = ffn // blk_f

    grid = (n_t, n_f)
    out, logits = pl.pallas_call(
        functools.partial(_moe_body, n_f),
        grid=grid,
        in_specs=[
            pl.BlockSpec((blk_t, hidden), lambda t, f: (t, 0)),
            pl.BlockSpec((n_experts, hidden), lambda t, f: (0, 0)),
            pl.BlockSpec((1, n_experts), lambda t, f: (0, 0)),
            pl.BlockSpec((blk_f, hidden), lambda t, f: (f, 0)),
            pl.BlockSpec((1, blk_f), lambda t, f: (0, f)),
            pl.BlockSpec((blk_f, hidden), lambda t, f: (f, 0)),
            pl.BlockSpec((1, blk_f), lambda t, f: (0, f)),
            pl.BlockSpec((hidden, blk_f), lambda t, f: (0, f)),
            pl.BlockSpec((1, hidden), lambda t, f: (0, 0)),
        ],
        out_specs=[
            pl.BlockSpec((blk_t, hidden), lambda t, f: (t, 0)),
            pl.BlockSpec((blk_t, n_experts), lambda t, f: (t, 0)),
        ],
        out_shape=[
            jax.ShapeDtypeStruct((n_tokens, hidden), jnp.float32),
            jax.ShapeDtypeStruct((n_tokens, n_experts), jnp.float32),
        ],
    )(h, gate_w, gate_b, w1_w, w1_b, w3_w, w3_b, w2_w, w2_b)
    return out, logits


def kernel(hidden_states, gate_w, gate_b, w1_w, w1_b, w3_w, w3_b, w2_w, w2_b):
    batch, seq, hidden = hidden_states.shape
    h = hidden_states.reshape(batch * seq, hidden)
    out, logits = _fused_moe(
        h, gate_w, gate_b.reshape(1, -1),
        w1_w, w1_b.reshape(1, -1),
        w3_w, w3_b.reshape(1, -1),
        w2_w, w2_b.reshape(1, -1),
        blk_t=512, blk_f=512,
    )
    return out.reshape(batch, seq, hidden), logits


# speedup vs baseline: 1.0113x; 1.0113x over previous
import functools

import jax
import jax.numpy as jnp
from jax.experimental import pallas as pl


def _moe_body(n_f, h_ref, gw_ref, gb_ref, w1_ref, b1_ref, w3_ref, b3_ref,
              w2_ref, b2_ref, out_ref, logits_ref):
    f = pl.program_id(1)
    h = h_ref[...]
    dn = (((1,), (1,)), ((), ()))
    a1 = jax.lax.dot_general(h, w1_ref[...], dn,
                             preferred_element_type=jnp.float32) + b1_ref[...]
    a3 = jax.lax.dot_general(h, w3_ref[...], dn,
                             preferred_element_type=jnp.float32) + b3_ref[...]
    act = (a1 * jax.nn.sigmoid(a1)) * a3
    partial = jax.lax.dot_general(act.astype(h.dtype), w2_ref[...], dn,
                                  preferred_element_type=jnp.float32)

    @pl.when(f == 0)
    def _init():
        logits_ref[...] = jax.lax.dot_general(
            h, gw_ref[...], dn, preferred_element_type=jnp.float32
        ) + gb_ref[...]
        out_ref[...] = partial + b2_ref[...]

    @pl.when(f != 0)
    def _accum():
        out_ref[...] += partial


def _fused_moe(h, gate_w, gate_b, w1_w, w1_b, w3_w, w3_b, w2_w, w2_b,
               blk_t, blk_f):
    n_tokens, hidden = h.shape
    ffn = w1_w.shape[0]
    n_experts = gate_w.shape[0]
    n_t = n_tokens // blk_t
    n_f = ffn // blk_f

    grid = (n_t, n_f)
    out, logits = pl.pallas_call(
        functools.partial(_moe_body, n_f),
        grid=grid,
        in_specs=[
            pl.BlockSpec((blk_t, hidden), lambda t, f: (t, 0)),
            pl.BlockSpec((n_experts, hidden), lambda t, f: (0, 0)),
            pl.BlockSpec((1, n_experts), lambda t, f: (0, 0)),
            pl.BlockSpec((blk_f, hidden), lambda t, f: (f, 0)),
            pl.BlockSpec((1, blk_f), lambda t, f: (0, f)),
            pl.BlockSpec((blk_f, hidden), lambda t, f: (f, 0)),
            pl.BlockSpec((1, blk_f), lambda t, f: (0, f)),
            pl.BlockSpec((hidden, blk_f), lambda t, f: (0, f)),
            pl.BlockSpec((1, hidden), lambda t, f: (0, 0)),
        ],
        out_specs=[
            pl.BlockSpec((blk_t, hidden), lambda t, f: (t, 0)),
            pl.BlockSpec((blk_t, n_experts), lambda t, f: (t, 0)),
        ],
        out_shape=[
            jax.ShapeDtypeStruct((n_tokens, hidden), jnp.float32),
            jax.ShapeDtypeStruct((n_tokens, n_experts), jnp.float32),
        ],
    )(h, gate_w, gate_b, w1_w, w1_b, w3_w, w3_b, w2_w, w2_b)
    return out, logits


def kernel(hidden_states, gate_w, gate_b, w1_w, w1_b, w3_w, w3_b, w2_w, w2_b):
    batch, seq, hidden = hidden_states.shape
    h = hidden_states.reshape(batch * seq, hidden).astype(jnp.bfloat16)
    out, logits = _fused_moe(
        h, gate_w.astype(jnp.bfloat16), gate_b.reshape(1, -1),
        w1_w.astype(jnp.bfloat16), w1_b.reshape(1, -1),
        w3_w.astype(jnp.bfloat16), w3_b.reshape(1, -1),
        w2_w.astype(jnp.bfloat16), w2_b.reshape(1, -1),
        blk_t=512, blk_f=512,
    )
    return out.reshape(batch, seq, hidden), logits


# speedup vs baseline: 1.1058x; 1.0935x over previous
import functools

import jax
import jax.numpy as jnp
from jax.experimental import pallas as pl
from jax.experimental.pallas import tpu as pltpu


def _moe_body(n_f, h_ref, gw_ref, gb_ref, w1_ref, b1_ref, w3_ref, b3_ref,
              w2_ref, w2_last_ref, b2_ref, out_ref, logits_ref, act_ref):
    f = pl.program_id(1)
    dn = (((1,), (1,)), ((), ()))
    h = h_ref[...]
    a1 = jax.lax.dot_general(h, w1_ref[...], dn,
                             preferred_element_type=jnp.float32) + b1_ref[...]
    a3 = jax.lax.dot_general(h, w3_ref[...], dn,
                             preferred_element_type=jnp.float32) + b3_ref[...]
    act = ((a1 * jax.nn.sigmoid(a1)) * a3).astype(h.dtype)

    slot = jax.lax.rem(f, 2)
    prev = jax.lax.rem(f + 1, 2)

    @pl.when(f == 0)
    def _gate():
        logits_ref[...] = jax.lax.dot_general(
            h, gw_ref[...], dn, preferred_element_type=jnp.float32
        ) + gb_ref[...]

    @pl.when(f == 1)
    def _first():
        out_ref[...] = jax.lax.dot_general(
            act_ref[prev], w2_ref[...], dn,
            preferred_element_type=jnp.float32) + b2_ref[...]

    @pl.when(f > 1)
    def _mid():
        out_ref[...] += jax.lax.dot_general(
            act_ref[prev], w2_ref[...], dn,
            preferred_element_type=jnp.float32)

    act_ref[slot] = act

    @pl.when(f == n_f - 1)
    def _last():
        out_ref[...] += jax.lax.dot_general(
            act, w2_last_ref[...], dn, preferred_element_type=jnp.float32)


def _fused_moe(h, gate_w, gate_b, w1_w, w1_b, w3_w, w3_b, w2_w, w2_b,
               blk_t, blk_f):
    n_tokens, hidden = h.shape
    ffn = w1_w.shape[0]
    n_experts = gate_w.shape[0]
    n_t = n_tokens // blk_t
    n_f = ffn // blk_f
    assert n_f >= 2

    grid = (n_t, n_f)
    lag = lambda t, f: (0, jnp.maximum(f - 1, 0))
    out, logits = pl.pallas_call(
        functools.partial(_moe_body, n_f),
        grid=grid,
        in_specs=[
            pl.BlockSpec((blk_t, hidden), lambda t, f: (t, 0)),
            pl.BlockSpec((n_experts, hidden), lambda t, f: (0, 0)),
            pl.BlockSpec((1, n_experts), lambda t, f: (0, 0)),
            pl.BlockSpec((blk_f, hidden), lambda t, f: (f, 0)),
            pl.BlockSpec((1, blk_f), lambda t, f: (0, f)),
            pl.BlockSpec((blk_f, hidden), lambda t, f: (f, 0)),
            pl.BlockSpec((1, blk_f), lambda t, f: (0, f)),
            pl.BlockSpec((hidden, blk_f), lag),
            pl.BlockSpec((hidden, blk_f), lambda t, f: (0, n_f - 1)),
            pl.BlockSpec((1, hidden), lambda t, f: (0, 0)),
        ],
        out_specs=[
            pl.BlockSpec((blk_t, hidden), lambda t, f: (t, 0)),
            pl.BlockSpec((blk_t, n_experts), lambda t, f: (t, 0)),
        ],
        out_shape=[
            jax.ShapeDtypeStruct((n_tokens, hidden), jnp.float32),
            jax.ShapeDtypeStruct((n_tokens, n_experts), jnp.float32),
        ],
        scratch_shapes=[pltpu.VMEM((2, blk_t, blk_f), h.dtype)],
        compiler_params=pltpu.CompilerParams(
            dimension_semantics=("parallel", "arbitrary")),
    )(h, gate_w, gate_b, w1_w, w1_b, w3_w, w3_b, w2_w, w2_w, w2_b)
    return out, logits


def kernel(hidden_states, gate_w, gate_b, w1_w, w1_b, w3_w, w3_b, w2_w, w2_b):
    batch, seq, hidden = hidden_states.shape
    h = hidden_states.reshape(batch * seq, hidden).astype(jnp.bfloat16)
    out, logits = _fused_moe(
        h, gate_w.astype(jnp.bfloat16), gate_b.reshape(1, -1),
        w1_w.astype(jnp.bfloat16), w1_b.reshape(1, -1),
        w3_w.astype(jnp.bfloat16), w3_b.reshape(1, -1),
        w2_w.astype(jnp.bfloat16), w2_b.reshape(1, -1),
        blk_t=512, blk_f=1024,
    )
    return out.reshape(batch, seq, hidden), logits


# speedup vs baseline: 1.1600x; 1.0490x over previous
import functools

import jax
import jax.numpy as jnp
from jax.experimental import pallas as pl
from jax.experimental.pallas import tpu as pltpu


def _moe_body(n_f, h_ref, gw_ref, gb_ref, w1_ref, b1_ref, w3_ref, b3_ref,
              w2_ref, b2_ref, out_ref, logits_ref):
    f = pl.program_id(1)
    dn = (((1,), (1,)), ((), ()))
    h = h_ref[...].astype(jnp.bfloat16)
    a1 = jax.lax.dot_general(h, w1_ref[...], dn,
                             preferred_element_type=jnp.float32) + b1_ref[...]
    a3 = jax.lax.dot_general(h, w3_ref[...], dn,
                             preferred_element_type=jnp.float32) + b3_ref[...]
    act = (a1 * jax.nn.sigmoid(a1)) * a3
    partial = jax.lax.dot_general(act.astype(jnp.bfloat16), w2_ref[...], dn,
                                  preferred_element_type=jnp.float32)

    @pl.when(f == 0)
    def _init():
        logits_ref[...] = jax.lax.dot_general(
            h, gw_ref[...], dn, preferred_element_type=jnp.float32
        ) + gb_ref[...]
        out_ref[...] = partial + b2_ref[...]

    @pl.when(f != 0)
    def _accum():
        out_ref[...] += partial


def _fused_moe(h, gate_w, gate_b, w1_w, w1_b, w3_w, w3_b, w2_w, w2_b,
               blk_t, blk_f):
    n_tokens, hidden = h.shape
    ffn = w1_w.shape[0]
    n_experts = gate_w.shape[0]
    n_t = n_tokens // blk_t
    n_f = ffn // blk_f

    grid = (n_t, n_f)
    out, logits = pl.pallas_call(
        functools.partial(_moe_body, n_f),
        grid=grid,
        in_specs=[
            pl.BlockSpec((blk_t, hidden), lambda t, f: (t, 0)),
            pl.BlockSpec((n_experts, hidden), lambda t, f: (0, 0)),
            pl.BlockSpec((1, n_experts), lambda t, f: (0, 0)),
            pl.BlockSpec((blk_f, hidden), lambda t, f: (f, 0)),
            pl.BlockSpec((1, blk_f), lambda t, f: (0, f)),
            pl.BlockSpec((blk_f, hidden), lambda t, f: (f, 0)),
            pl.BlockSpec((1, blk_f), lambda t, f: (0, f)),
            pl.BlockSpec((hidden, blk_f), lambda t, f: (0, f)),
            pl.BlockSpec((1, hidden), lambda t, f: (0, 0)),
        ],
        out_specs=[
            pl.BlockSpec((blk_t, hidden), lambda t, f: (t, 0)),
            pl.BlockSpec((blk_t, n_experts), lambda t, f: (t, 0)),
        ],
        out_shape=[
            jax.ShapeDtypeStruct((n_tokens, hidden), jnp.float32),
            jax.ShapeDtypeStruct((n_tokens, n_experts), jnp.float32),
        ],
        compiler_params=pltpu.CompilerParams(
            dimension_semantics=("parallel", "arbitrary")),
    )(h, gate_w, gate_b, w1_w, w1_b, w3_w, w3_b, w2_w, w2_b)
    return out, logits


def kernel(hidden_states, gate_w, gate_b, w1_w, w1_b, w3_w, w3_b, w2_w, w2_b):
    batch, seq, hidden = hidden_states.shape
    h = hidden_states.reshape(batch * seq, hidden)
    out, logits = _fused_moe(
        h, gate_w.astype(jnp.bfloat16), gate_b.reshape(1, -1),
        w1_w.astype(jnp.bfloat16), w1_b.reshape(1, -1),
        w3_w.astype(jnp.bfloat16), w3_b.reshape(1, -1),
        w2_w.astype(jnp.bfloat16), w2_b.reshape(1, -1),
        blk_t=512, blk_f=1024,
    )
    return out.reshape(batch, seq, hidden), logits
